# baseline (speedup 1.0000x reference)
import functools

import jax
import jax.numpy as jnp
from jax import lax
from jax.experimental import pallas as pl
from jax.experimental.pallas import tpu as pltpu
from jax.experimental.pallas import tpu_sc as plsc

B, H, W, K = 4, 384, 384, 8
P = B * H * W
NW = 32
G = W
NG = G // 16
TROWS = (B * H) // NW
NJ = TROWS // 2

MAGIC = 2.0 ** 23


def _round_half_even(x):
    return (x + MAGIC) - MAGIC


@functools.partial(
    pl.kernel,
    mesh=plsc.VectorSubcoreMesh(core_axis_name="c", subcore_axis_name="s"),
    compiler_params=pltpu.CompilerParams(needs_layout_passes=False),
    out_type=(
        jax.ShapeDtypeStruct((4, P), jnp.float32),
        jax.ShapeDtypeStruct((B * H, W), jnp.float32),
        jax.ShapeDtypeStruct((B * H, W), jnp.int32),
        jax.ShapeDtypeStruct((B * H * K, 4, W), jnp.float32),
    ),
    scratch_types=[
        pltpu.VMEM((5 * K, W), jnp.float32),
        pltpu.VMEM((5 * K, W), jnp.float32),
        pltpu.VMEM((K, W), jnp.float32),
        pltpu.VMEM((K, W), jnp.float32),
        pltpu.VMEM((4, W), jnp.float32),
        pltpu.VMEM((4, W), jnp.float32),
        pltpu.VMEM((K, 4, W), jnp.float32),
        pltpu.VMEM((K, 4, W), jnp.float32),
        pltpu.VMEM((TROWS, W), jnp.float32),
        pltpu.VMEM((TROWS, W), jnp.int32),
        pltpu.SemaphoreType.DMA,
        pltpu.SemaphoreType.DMA,
        pltpu.SemaphoreType.DMA,
        pltpu.SemaphoreType.DMA,
        pltpu.SemaphoreType.DMA,
        pltpu.SemaphoreType.DMA,
        pltpu.SemaphoreType.DMA,
        pltpu.SemaphoreType.DMA,
    ],
)
def _sc_shader(st_h, z_h, comp_h, depth_h, lab_h, hum_h,
               st_a, st_b, z_a, z_b, comp_a, comp_b, hum_a, hum_b,
               depth_v, lab_v,
               s_st_a, s_st_b, s_z_a, s_z_b,
               s_comp_a, s_comp_b, s_hum_a, s_hum_b):
    wid = lax.axis_index("s") * 2 + lax.axis_index("c")
    lane = lax.iota(jnp.int32, 16)
    row0 = pl.multiple_of(wid * TROWS, 8)

    def start_in(c, st_v, z_v, s_st, s_z):
        pltpu.async_copy(st_h.at[pl.ds(pl.multiple_of(c * (5 * K), 8), 5 * K)], st_v, s_st)
        pltpu.async_copy(z_h.at[pl.ds(pl.multiple_of(c * K, 8), K)], z_v, s_z)

    def wait_in(st_v, z_v, s_st, s_z):
        pltpu.make_async_copy(st_h.at[pl.ds(0, 5 * K)], st_v, s_st).wait()
        pltpu.make_async_copy(z_h.at[pl.ds(0, K)], z_v, s_z).wait()

    def fire_out(c, comp_v, hum_v, s_comp, s_hum):
        pltpu.async_copy(comp_v, comp_h.at[:, pl.ds(pl.multiple_of(c * W, 128), W)], s_comp)
        pltpu.async_copy(hum_v, hum_h.at[pl.ds(pl.multiple_of(c * K, 8), K)], s_hum)

    def wait_out(comp_v, hum_v, s_comp, s_hum):
        pltpu.make_async_copy(comp_v, comp_h.at[:, pl.ds(0, W)], s_comp).wait()
        pltpu.make_async_copy(hum_v, hum_h.at[pl.ds(0, K)], s_hum).wait()

    def compute(tr, st_v, z_v, comp_v, hum_v):
        @plsc.parallel_loop(0, NG, 1, unroll=8)
        def group(g):
            col = g * 16
            pcol = lane + col
            one = jnp.full((16,), 1.0, jnp.float32)
            for n in range(K):
                hum_v[n, 0, pl.ds(col, 16)] = one
                hum_v[n, 1, pl.ds(col, 16)] = one
                hum_v[n, 2, pl.ds(col, 16)] = one
                hum_v[n, 3, pl.ds(col, 16)] = one * 0.0
            rgb0 = one
            rgb1 = one
            rgb2 = one
            aacc = jnp.zeros((16,), jnp.float32)
            depth = jnp.full((16,), 100.0, jnp.float32)
            labf = jnp.full((16,), float(K), jnp.float32)
            zero = lane * 0
            for k in range(K - 1, -1, -1):
                c0 = st_v[0 * K + k, pl.ds(col, 16)]
                c1 = st_v[1 * K + k, pl.ds(col, 16)]
                c2v = st_v[2 * K + k, pl.ds(col, 16)]
                a = st_v[3 * K + k, pl.ds(col, 16)]
                labk = st_v[4 * K + k, pl.ds(col, 16)]
                z = z_v[k, pl.ds(col, 16)]
                om = 1.0 - a
                rgb0 = c0 * a + rgb0 * om
                rgb1 = c1 * a + rgb1 * om
                rgb2 = c2v * a + rgb2 * om
                aacc = jnp.maximum(a, aacc)
                zvalid = z >= 0.0
                depth = jnp.where(z > 0.0, z * a + depth * om, depth)
                labf = jnp.where(zvalid & (a > 0.5), labk, labf)
                li = _round_half_even(labk).astype(jnp.int32)
                m = zvalid
                plsc.store_scatter(hum_v, [li, zero, pcol], c0 * a + om, mask=m)
                plsc.store_scatter(hum_v, [li, zero + 1, pcol], c1 * a + om, mask=m)
                plsc.store_scatter(hum_v, [li, zero + 2, pcol], c2v * a + om, mask=m)
                plsc.store_scatter(hum_v, [li, zero + 3, pcol], a, mask=m)
            comp_v[0, pl.ds(col, 16)] = rgb0
            comp_v[1, pl.ds(col, 16)] = rgb1
            comp_v[2, pl.ds(col, 16)] = rgb2
            comp_v[3, pl.ds(col, 16)] = aacc
            depth_v[tr, pl.ds(col, 16)] = depth
            labf2 = jnp.where(labf > K - 0.5, jnp.float32(-1.0), labf)
            lab_v[tr, pl.ds(col, 16)] = _round_half_even(labf2).astype(jnp.int32)

    start_in(row0, st_a, z_a, s_st_a, s_z_a)

    def body(j, carry):
        c0 = row0 + 2 * j
        c1 = c0 + 1
        start_in(c1, st_b, z_b, s_st_b, s_z_b)
        wait_in(st_a, z_a, s_st_a, s_z_a)
        pl.when(j > 0)(lambda: wait_out(comp_a, hum_a, s_comp_a, s_hum_a))
        compute(2 * j, st_a, z_a, comp_a, hum_a)
        fire_out(c0, comp_a, hum_a, s_comp_a, s_hum_a)
        pl.when(j < NJ - 1)(lambda: start_in(c0 + 2, st_a, z_a, s_st_a, s_z_a))
        wait_in(st_b, z_b, s_st_b, s_z_b)
        pl.when(j > 0)(lambda: wait_out(comp_b, hum_b, s_comp_b, s_hum_b))
        compute(2 * j + 1, st_b, z_b, comp_b, hum_b)
        fire_out(c1, comp_b, hum_b, s_comp_b, s_hum_b)
        return carry

    lax.fori_loop(0, NJ, body, 0)
    wait_out(comp_a, hum_a, s_comp_a, s_hum_a)
    wait_out(comp_b, hum_b, s_comp_b, s_hum_b)
    pltpu.sync_copy(depth_v, depth_h.at[pl.ds(row0, TROWS)])
    pltpu.sync_copy(lab_v, lab_h.at[pl.ds(row0, TROWS)])


def kernel(sampled_textures, zbuf):
    st_t = jnp.transpose(sampled_textures, (0, 1, 4, 3, 2)).reshape(B * H * 5 * K, W)
    z_t = jnp.transpose(zbuf, (0, 1, 3, 2)).reshape(B * H * K, W)
    comp_t, depth, lab, hum_t = _sc_shader(st_t, z_t)
    composite_image = jnp.transpose(comp_t.reshape(4, B, H, W), (1, 2, 3, 0))
    composite_depth = depth.reshape(B, H, W)
    composite_label = lab.reshape(B, H, W).astype(jnp.int64)
    human_images = jnp.transpose(hum_t.reshape(B, H, K, 4, W), (0, 1, 4, 2, 3))
    return composite_image, composite_depth, composite_label, human_images


# speedup vs baseline: 1.2890x; 1.2890x over previous
import functools

import jax
import jax.numpy as jnp
from jax import lax
from jax.experimental import pallas as pl
from jax.experimental.pallas import tpu as pltpu
from jax.experimental.pallas import tpu_sc as plsc

B, H, W, K = 4, 384, 384, 8
P = B * H * W
NW = 32
G = W
NG = G // 16
TROWS = (B * H) // NW
NJ = TROWS // 2

MAGIC = 2.0 ** 23


def _round_half_even(x):
    return (x + MAGIC) - MAGIC


@functools.partial(
    pl.kernel,
    mesh=plsc.VectorSubcoreMesh(core_axis_name="c", subcore_axis_name="s"),
    compiler_params=pltpu.CompilerParams(needs_layout_passes=False),
    out_type=(
        jax.ShapeDtypeStruct((4, P), jnp.float32),
        jax.ShapeDtypeStruct((B * H, W), jnp.float32),
        jax.ShapeDtypeStruct((B * H, W), jnp.int32),
        jax.ShapeDtypeStruct((B * H * K, 4, W), jnp.float32),
    ),
    scratch_types=[
        pltpu.VMEM((5 * K, W), jnp.float32),
        pltpu.VMEM((5 * K, W), jnp.float32),
        pltpu.VMEM((K, W), jnp.float32),
        pltpu.VMEM((K, W), jnp.float32),
        pltpu.VMEM((4, W), jnp.float32),
        pltpu.VMEM((4, W), jnp.float32),
        pltpu.VMEM((K, 4, W), jnp.float32),
        pltpu.VMEM((K, 4, W), jnp.float32),
        pltpu.VMEM((TROWS, W), jnp.float32),
        pltpu.VMEM((TROWS, W), jnp.int32),
        pltpu.SemaphoreType.DMA,
        pltpu.SemaphoreType.DMA,
        pltpu.SemaphoreType.DMA,
        pltpu.SemaphoreType.DMA,
        pltpu.SemaphoreType.DMA,
        pltpu.SemaphoreType.DMA,
        pltpu.SemaphoreType.DMA,
        pltpu.SemaphoreType.DMA,
    ],
)
def _sc_shader(st_h, z_h, comp_h, depth_h, lab_h, hum_h,
               st_a, st_b, z_a, z_b, comp_a, comp_b, hum_a, hum_b,
               depth_v, lab_v,
               s_st_a, s_st_b, s_z_a, s_z_b,
               s_comp_a, s_comp_b, s_hum_a, s_hum_b):
    wid = lax.axis_index("s") * 2 + lax.axis_index("c")
    lane = lax.iota(jnp.int32, 16)
    row0 = pl.multiple_of(wid * TROWS, 8)

    def start_in(c, st_v, z_v, s_st, s_z):
        pltpu.async_copy(st_h.at[pl.ds(pl.multiple_of(c * (5 * K), 8), 5 * K)], st_v, s_st)
        pltpu.async_copy(z_h.at[pl.ds(pl.multiple_of(c * K, 8), K)], z_v, s_z)

    def wait_in(st_v, z_v, s_st, s_z):
        pltpu.make_async_copy(st_h.at[pl.ds(0, 5 * K)], st_v, s_st).wait()
        pltpu.make_async_copy(z_h.at[pl.ds(0, K)], z_v, s_z).wait()

    def fire_out(c, comp_v, hum_v, s_comp, s_hum):
        pltpu.async_copy(comp_v, comp_h.at[:, pl.ds(pl.multiple_of(c * W, 128), W)], s_comp)
        pltpu.async_copy(hum_v, hum_h.at[pl.ds(pl.multiple_of(c * K, 8), K)], s_hum)

    def wait_out(comp_v, hum_v, s_comp, s_hum):
        pltpu.make_async_copy(comp_v, comp_h.at[:, pl.ds(0, W)], s_comp).wait()
        pltpu.make_async_copy(hum_v, hum_h.at[pl.ds(0, K)], s_hum).wait()

    def compute(tr, st_v, z_v, comp_v, hum_v):
        @plsc.parallel_loop(0, NG, 1, unroll=6)
        def group(g):
            col = g * 16
            pcol = lane + col
            one = jnp.full((16,), 1.0, jnp.float32)
            for n in range(K):
                hum_v[n, 0, pl.ds(col, 16)] = one
                hum_v[n, 1, pl.ds(col, 16)] = one
                hum_v[n, 2, pl.ds(col, 16)] = one
                hum_v[n, 3, pl.ds(col, 16)] = one * 0.0
            rgb0 = one
            rgb1 = one
            rgb2 = one
            aacc = jnp.zeros((16,), jnp.float32)
            depth = jnp.full((16,), 100.0, jnp.float32)
            labf = jnp.full((16,), float(K), jnp.float32)
            zero = lane * 0
            for k in range(K - 1, -1, -1):
                c0 = st_v[0 * K + k, pl.ds(col, 16)]
                c1 = st_v[1 * K + k, pl.ds(col, 16)]
                c2v = st_v[2 * K + k, pl.ds(col, 16)]
                a = st_v[3 * K + k, pl.ds(col, 16)]
                labk = st_v[4 * K + k, pl.ds(col, 16)]
                z = z_v[k, pl.ds(col, 16)]
                om = 1.0 - a
                rgb0 = c0 * a + rgb0 * om
                rgb1 = c1 * a + rgb1 * om
                rgb2 = c2v * a + rgb2 * om
                aacc = jnp.maximum(a, aacc)
                zvalid = z >= 0.0
                depth = jnp.where(z > 0.0, z * a + depth * om, depth)
                labf = jnp.where(zvalid & (a > 0.5), labk, labf)
                li = _round_half_even(labk).astype(jnp.int32)
                m = zvalid
                plsc.store_scatter(hum_v, [li, zero, pcol], c0 * a + om, mask=m)
                plsc.store_scatter(hum_v, [li, zero + 1, pcol], c1 * a + om, mask=m)
                plsc.store_scatter(hum_v, [li, zero + 2, pcol], c2v * a + om, mask=m)
                plsc.store_scatter(hum_v, [li, zero + 3, pcol], a, mask=m)
            comp_v[0, pl.ds(col, 16)] = rgb0
            comp_v[1, pl.ds(col, 16)] = rgb1
            comp_v[2, pl.ds(col, 16)] = rgb2
            comp_v[3, pl.ds(col, 16)] = aacc
            depth_v[tr, pl.ds(col, 16)] = depth
            labf2 = jnp.where(labf > K - 0.5, jnp.float32(-1.0), labf)
            lab_v[tr, pl.ds(col, 16)] = _round_half_even(labf2).astype(jnp.int32)

    start_in(row0, st_a, z_a, s_st_a, s_z_a)

    def body(j, carry):
        c0 = row0 + 2 * j
        c1 = c0 + 1
        start_in(c1, st_b, z_b, s_st_b, s_z_b)
        wait_in(st_a, z_a, s_st_a, s_z_a)
        pl.when(j > 0)(lambda: wait_out(comp_a, hum_a, s_comp_a, s_hum_a))
        compute(2 * j, st_a, z_a, comp_a, hum_a)
        fire_out(c0, comp_a, hum_a, s_comp_a, s_hum_a)
        pl.when(j < NJ - 1)(lambda: start_in(c0 + 2, st_a, z_a, s_st_a, s_z_a))
        wait_in(st_b, z_b, s_st_b, s_z_b)
        pl.when(j > 0)(lambda: wait_out(comp_b, hum_b, s_comp_b, s_hum_b))
        compute(2 * j + 1, st_b, z_b, comp_b, hum_b)
        fire_out(c1, comp_b, hum_b, s_comp_b, s_hum_b)
        return carry

    lax.fori_loop(0, NJ, body, 0)
    wait_out(comp_a, hum_a, s_comp_a, s_hum_a)
    wait_out(comp_b, hum_b, s_comp_b, s_hum_b)
    pltpu.sync_copy(depth_v, depth_h.at[pl.ds(row0, TROWS)])
    pltpu.sync_copy(lab_v, lab_h.at[pl.ds(row0, TROWS)])


def kernel(sampled_textures, zbuf):
    st_t = jnp.transpose(sampled_textures, (0, 1, 4, 3, 2)).reshape(B * H * 5 * K, W)
    z_t = jnp.transpose(zbuf, (0, 1, 3, 2)).reshape(B * H * K, W)
    comp_t, depth, lab, hum_t = _sc_shader(st_t, z_t)
    composite_image = jnp.transpose(comp_t.reshape(4, B, H, W), (1, 2, 3, 0))
    composite_depth = depth.reshape(B, H, W)
    composite_label = lab.reshape(B, H, W).astype(jnp.int64)
    human_images = jnp.transpose(hum_t.reshape(B, H, K, 4, W), (0, 1, 4, 2, 3))
    return composite_image, composite_depth, composite_label, human_images


# speedup vs baseline: 1.6434x; 1.2749x over previous
import functools

import jax
import jax.numpy as jnp
from jax import lax
from jax.experimental import pallas as pl
from jax.experimental.pallas import tpu as pltpu
from jax.experimental.pallas import tpu_sc as plsc

B, H, W, K = 4, 384, 384, 8
P = B * H * W
NW = 32
G = W
NG = G // 16
TROWS = (B * H) // NW
NJ = TROWS // 2

MAGIC = 2.0 ** 23


def _round_half_even(x):
    return (x + MAGIC) - MAGIC


@functools.partial(
    pl.kernel,
    mesh=plsc.VectorSubcoreMesh(core_axis_name="c", subcore_axis_name="s"),
    compiler_params=pltpu.CompilerParams(needs_layout_passes=False),
    out_type=(
        jax.ShapeDtypeStruct((4, P), jnp.float32),
        jax.ShapeDtypeStruct((B * H, W), jnp.float32),
        jax.ShapeDtypeStruct((B * H, W), jnp.int32),
        jax.ShapeDtypeStruct((B * H * K, 4, W), jnp.float32),
    ),
    scratch_types=[
        pltpu.VMEM((5 * K, W), jnp.float32),
        pltpu.VMEM((5 * K, W), jnp.float32),
        pltpu.VMEM((K, W), jnp.float32),
        pltpu.VMEM((K, W), jnp.float32),
        pltpu.VMEM((4, W), jnp.float32),
        pltpu.VMEM((4, W), jnp.float32),
        pltpu.VMEM((K, 4, W), jnp.float32),
        pltpu.VMEM((K, 4, W), jnp.float32),
        pltpu.VMEM((TROWS, W), jnp.float32),
        pltpu.VMEM((TROWS, W), jnp.int32),
        pltpu.SemaphoreType.DMA,
        pltpu.SemaphoreType.DMA,
        pltpu.SemaphoreType.DMA,
        pltpu.SemaphoreType.DMA,
        pltpu.SemaphoreType.DMA,
        pltpu.SemaphoreType.DMA,
        pltpu.SemaphoreType.DMA,
        pltpu.SemaphoreType.DMA,
    ],
)
def _sc_shader(st_h, z_h, comp_h, depth_h, lab_h, hum_h,
               st_a, st_b, z_a, z_b, comp_a, comp_b, hum_a, hum_b,
               depth_v, lab_v,
               s_st_a, s_st_b, s_z_a, s_z_b,
               s_comp_a, s_comp_b, s_hum_a, s_hum_b):
    wid = lax.axis_index("s") * 2 + lax.axis_index("c")
    lane = lax.iota(jnp.int32, 16)
    row0 = pl.multiple_of(wid * TROWS, 8)

    def start_in(c, st_v, z_v, s_st, s_z):
        pltpu.async_copy(st_h.at[pl.ds(pl.multiple_of(c * (5 * K), 8), 5 * K)], st_v, s_st)
        pltpu.async_copy(z_h.at[pl.ds(pl.multiple_of(c * K, 8), K)], z_v, s_z)

    def wait_in(st_v, z_v, s_st, s_z):
        pltpu.make_async_copy(st_h.at[pl.ds(0, 5 * K)], st_v, s_st).wait()
        pltpu.make_async_copy(z_h.at[pl.ds(0, K)], z_v, s_z).wait()

    def fire_out(c, comp_v, hum_v, s_comp, s_hum):
        pltpu.async_copy(comp_v, comp_h.at[:, pl.ds(pl.multiple_of(c * W, 128), W)], s_comp)
        pltpu.async_copy(hum_v, hum_h.at[pl.ds(pl.multiple_of(c * K, 8), K)], s_hum)

    def wait_out(comp_v, hum_v, s_comp, s_hum):
        pltpu.make_async_copy(comp_v, comp_h.at[:, pl.ds(0, W)], s_comp).wait()
        pltpu.make_async_copy(hum_v, hum_h.at[pl.ds(0, K)], s_hum).wait()

    def compute(tr, st_v, z_v, comp_v, hum_v):
        @plsc.parallel_loop(0, NG, 1, unroll=4)
        def group(g):
            col = g * 16
            pcol = lane + col
            one = jnp.full((16,), 1.0, jnp.float32)
            for n in range(K):
                hum_v[n, 0, pl.ds(col, 16)] = one
                hum_v[n, 1, pl.ds(col, 16)] = one
                hum_v[n, 2, pl.ds(col, 16)] = one
                hum_v[n, 3, pl.ds(col, 16)] = one * 0.0
            rgb0 = one
            rgb1 = one
            rgb2 = one
            aacc = jnp.zeros((16,), jnp.float32)
            depth = jnp.full((16,), 100.0, jnp.float32)
            labf = jnp.full((16,), float(K), jnp.float32)
            zero = lane * 0
            for k in range(K - 1, -1, -1):
                c0 = st_v[0 * K + k, pl.ds(col, 16)]
                c1 = st_v[1 * K + k, pl.ds(col, 16)]
                c2v = st_v[2 * K + k, pl.ds(col, 16)]
                a = st_v[3 * K + k, pl.ds(col, 16)]
                labk = st_v[4 * K + k, pl.ds(col, 16)]
                z = z_v[k, pl.ds(col, 16)]
                om = 1.0 - a
                rgb0 = c0 * a + rgb0 * om
                rgb1 = c1 * a + rgb1 * om
                rgb2 = c2v * a + rgb2 * om
                aacc = jnp.maximum(a, aacc)
                zvalid = z >= 0.0
                depth = jnp.where(z > 0.0, z * a + depth * om, depth)
                labf = jnp.where(zvalid & (a > 0.5), labk, labf)
                li = _round_half_even(labk).astype(jnp.int32)
                m = zvalid
                plsc.store_scatter(hum_v, [li, zero, pcol], c0 * a + om, mask=m)
                plsc.store_scatter(hum_v, [li, zero + 1, pcol], c1 * a + om, mask=m)
                plsc.store_scatter(hum_v, [li, zero + 2, pcol], c2v * a + om, mask=m)
                plsc.store_scatter(hum_v, [li, zero + 3, pcol], a, mask=m)
            comp_v[0, pl.ds(col, 16)] = rgb0
            comp_v[1, pl.ds(col, 16)] = rgb1
            comp_v[2, pl.ds(col, 16)] = rgb2
            comp_v[3, pl.ds(col, 16)] = aacc
            depth_v[tr, pl.ds(col, 16)] = depth
            labf2 = jnp.where(labf > K - 0.5, jnp.float32(-1.0), labf)
            lab_v[tr, pl.ds(col, 16)] = _round_half_even(labf2).astype(jnp.int32)

    start_in(row0, st_a, z_a, s_st_a, s_z_a)

    def body(j, carry):
        c0 = row0 + 2 * j
        c1 = c0 + 1
        start_in(c1, st_b, z_b, s_st_b, s_z_b)
        wait_in(st_a, z_a, s_st_a, s_z_a)
        pl.when(j > 0)(lambda: wait_out(comp_a, hum_a, s_comp_a, s_hum_a))
        compute(2 * j, st_a, z_a, comp_a, hum_a)
        fire_out(c0, comp_a, hum_a, s_comp_a, s_hum_a)
        pl.when(j < NJ - 1)(lambda: start_in(c0 + 2, st_a, z_a, s_st_a, s_z_a))
        wait_in(st_b, z_b, s_st_b, s_z_b)
        pl.when(j > 0)(lambda: wait_out(comp_b, hum_b, s_comp_b, s_hum_b))
        compute(2 * j + 1, st_b, z_b, comp_b, hum_b)
        fire_out(c1, comp_b, hum_b, s_comp_b, s_hum_b)
        return carry

    lax.fori_loop(0, NJ, body, 0)
    wait_out(comp_a, hum_a, s_comp_a, s_hum_a)
    wait_out(comp_b, hum_b, s_comp_b, s_hum_b)
    pltpu.sync_copy(depth_v, depth_h.at[pl.ds(row0, TROWS)])
    pltpu.sync_copy(lab_v, lab_h.at[pl.ds(row0, TROWS)])


def kernel(sampled_textures, zbuf):
    st_t = jnp.transpose(sampled_textures, (0, 1, 4, 3, 2)).reshape(B * H * 5 * K, W)
    z_t = jnp.transpose(zbuf, (0, 1, 3, 2)).reshape(B * H * K, W)
    comp_t, depth, lab, hum_t = _sc_shader(st_t, z_t)
    composite_image = jnp.transpose(comp_t.reshape(4, B, H, W), (1, 2, 3, 0))
    composite_depth = depth.reshape(B, H, W)
    composite_label = lab.reshape(B, H, W).astype(jnp.int64)
    human_images = jnp.transpose(hum_t.reshape(B, H, K, 4, W), (0, 1, 4, 2, 3))
    return composite_image, composite_depth, composite_label, human_images
